# baseline (speedup 1.0000x reference)
import functools

import jax
import jax.numpy as jnp
from jax import lax
from jax.experimental import pallas as pl
from jax.experimental.pallas import tpu as pltpu
from jax.experimental.pallas import tpu_sc as plsc

BATCH = 16384
EMB_K = 64
NUM_CORES = 2
NUM_SUBCORES = 16
NW = NUM_CORES * NUM_SUBCORES
BPW = BATCH // NW
GROUPS = BPW // 16
WLEN = 144


def _run(uidx_hbm, iidx_hbm, ut_hbm, it_hbm, w_hbm, out_hbm,
         uidx_v, iidx_v, urows_v, irows_v, w_v, out_v, sem_u, sem_i):
    wid = lax.axis_index("s") * NUM_CORES + lax.axis_index("c")
    base = wid * BPW
    pltpu.sync_copy(uidx_hbm.at[pl.ds(base, BPW)], uidx_v)
    pltpu.sync_copy(iidx_hbm.at[pl.ds(base, BPW)], iidx_v)
    cu = pltpu.async_copy(ut_hbm.at[uidx_v], urows_v, sem_u)
    ci = pltpu.async_copy(it_hbm.at[iidx_v], irows_v, sem_i)
    pltpu.sync_copy(w_hbm, w_v)
    cu.wait()
    ci.wait()

    wvecs = [w_v[pl.ds(j * 16, 16)] for j in range(WLEN // 16)]
    wu = [wvecs[k // 16][k % 16] for k in range(EMB_K)]
    wi = [wvecs[(EMB_K + k) // 16][k % 16] for k in range(EMB_K)]
    bias = wvecs[(2 * EMB_K) // 16][0]
    lane = lax.iota(jnp.int32, 16)
    cols = [jnp.full((16,), k, jnp.int32) for k in range(EMB_K)]

    def body(g, carry):
        rows = lane + g * 16
        acc = jnp.full((16,), 0.0, jnp.float32) + bias
        for k in range(EMB_K):
            ucol = plsc.load_gather(urows_v, [rows, cols[k]])
            icol = plsc.load_gather(irows_v, [rows, cols[k]])
            acc = acc + ucol * wu[k] + icol * wi[k]
        out_v[pl.ds(g * 16, 16)] = 1.0 / (1.0 + jnp.exp(-acc))
        return carry

    lax.fori_loop(0, GROUPS, body, 0)
    pltpu.sync_copy(out_v, out_hbm.at[pl.ds(base, BPW)])


@jax.jit
def _launch(u_idx, i_idx, user_table, item_table, wb):
    mesh = plsc.VectorSubcoreMesh(
        core_axis_name="c", subcore_axis_name="s",
        num_cores=NUM_CORES, num_subcores=NUM_SUBCORES)
    kern = functools.partial(
        pl.kernel,
        out_type=jax.ShapeDtypeStruct((BATCH,), jnp.float32),
        mesh=mesh,
        compiler_params=pltpu.CompilerParams(
            needs_layout_passes=False, use_tc_tiling_on_sc=False),
        scratch_types=[
            pltpu.VMEM((BPW,), jnp.int32),
            pltpu.VMEM((BPW,), jnp.int32),
            pltpu.VMEM((BPW, EMB_K), jnp.float32),
            pltpu.VMEM((BPW, EMB_K), jnp.float32),
            pltpu.VMEM((WLEN,), jnp.float32),
            pltpu.VMEM((BPW,), jnp.float32),
            pltpu.SemaphoreType.DMA,
            pltpu.SemaphoreType.DMA,
        ],
    )(_run)
    return kern(u_idx, i_idx, user_table, item_table, wb)


def kernel(x, user_table, item_table, W, b):
    u_idx = x[:, 0].astype(jnp.int32)
    i_idx = x[:, 1].astype(jnp.int32)
    wb = jnp.concatenate(
        [W.reshape(-1), b.reshape(-1),
         jnp.zeros((WLEN - 2 * EMB_K - 1,), jnp.float32)])
    return _launch(u_idx, i_idx, user_table, item_table, wb)
